# Initial kernel scaffold; baseline (speedup 1.0000x reference)
import jax
import jax.numpy as jnp
from jax.experimental import pallas as pl


def kernel(indices, table):
    raise NotImplementedError("write your pallas kernel here")


# speedup vs baseline: 1.0232x; 1.0232x over previous
import functools

import jax
import jax.numpy as jnp
from jax import lax
from jax.experimental import pallas as pl
from jax.experimental.pallas import tpu as pltpu
from jax.experimental.pallas import tpu_sc as plsc

D = 32
CHUNK = 128
B, L = 16384, 50

_info = plsc.get_sparse_core_info()
NC, NS = _info.num_cores, _info.num_subcores
NW = NC * NS


@functools.lru_cache(maxsize=None)
def _make_gather(rows_per_w: int, vocab: int):
    tot = NW * rows_per_w * CHUNK
    mesh = plsc.VectorSubcoreMesh(core_axis_name="c", subcore_axis_name="s")

    @functools.partial(
        pl.kernel,
        mesh=mesh,
        out_type=jax.ShapeDtypeStruct((tot, D), jnp.float32),
        scratch_types=[
            pltpu.VMEM((rows_per_w, CHUNK), jnp.int32),
            pltpu.VMEM((CHUNK, D), jnp.float32),
            pltpu.SemaphoreType.DMA,
        ],
        compiler_params=pltpu.CompilerParams(use_tc_tiling_on_sc=False),
    )
    def body(idx_hbm, table_hbm, out_hbm, idx_v, rows_v, gsem):
        wid = lax.axis_index("s") * NC + lax.axis_index("c")
        base = wid * rows_per_w * CHUNK
        pltpu.sync_copy(idx_hbm.at[wid], idx_v)

        def step(g, carry):
            pltpu.async_copy(table_hbm.at[idx_v.at[g]], rows_v, gsem).wait()
            pltpu.sync_copy(
                rows_v, out_hbm.at[pl.ds(base + g * CHUNK, CHUNK)])
            return carry

        lax.fori_loop(0, rows_per_w, step, 0)

    return body


def kernel(indices, table):
    tot = indices.shape[0] * indices.shape[1]
    rows_per_w = tot // (NW * CHUNK)
    idx = indices.reshape(NW, rows_per_w, CHUNK)
    out = _make_gather(rows_per_w, table.shape[0])(idx, table)
    return out.reshape(indices.shape[0], indices.shape[1], D)


# speedup vs baseline: 1.1139x; 1.0886x over previous
import functools

import jax
import jax.numpy as jnp
from jax import lax
from jax.experimental import pallas as pl
from jax.experimental.pallas import tpu as pltpu
from jax.experimental.pallas import tpu_sc as plsc

D = 32
CHUNK = 128
K = 10
GROUP = K * CHUNK

_info = plsc.get_sparse_core_info()
NC, NS = _info.num_cores, _info.num_subcores
NW = NC * NS


@functools.lru_cache(maxsize=None)
def _make_gather(rows_per_w: int, vocab: int):
    tot = NW * rows_per_w * CHUNK
    groups = rows_per_w // K
    assert groups % 2 == 0 and groups >= 4
    mesh = plsc.VectorSubcoreMesh(core_axis_name="c", subcore_axis_name="s")

    @functools.partial(
        pl.kernel,
        mesh=mesh,
        out_type=jax.ShapeDtypeStruct((tot, D), jnp.float32),
        scratch_types=[
            pltpu.VMEM((rows_per_w, CHUNK), jnp.int32),
            pltpu.VMEM((2, GROUP, D), jnp.float32),
            pltpu.SemaphoreType.DMA,
            pltpu.SemaphoreType.DMA,
            pltpu.SemaphoreType.DMA,
            pltpu.SemaphoreType.DMA,
        ],
        compiler_params=pltpu.CompilerParams(use_tc_tiling_on_sc=False),
    )
    def body(idx_hbm, table_hbm, out_hbm, idx_v, rows_v, gs0, gs1, ws0, ws1):
        wid = lax.axis_index("s") * NC + lax.axis_index("c")
        base = wid * rows_per_w * CHUNK
        pltpu.sync_copy(idx_hbm.at[wid], idx_v)

        gsem = (gs0, gs1)
        wsem = (ws0, ws1)

        def fire_gathers(g, b):
            for j in range(K):
                pltpu.async_copy(
                    table_hbm.at[idx_v.at[g * K + j]],
                    rows_v.at[b].at[pl.ds(j * CHUNK, CHUNK)],
                    gsem[b])

        def wait_gathers(b):
            pltpu.make_async_copy(
                table_hbm.at[pl.ds(0, GROUP)], rows_v.at[b], gsem[b]).wait()

        def fire_write(g, b):
            pltpu.async_copy(
                rows_v.at[b], out_hbm.at[pl.ds(base + g * GROUP, GROUP)],
                wsem[b])

        def wait_write(b):
            pltpu.make_async_copy(
                rows_v.at[b], out_hbm.at[pl.ds(base, GROUP)], wsem[b]).wait()

        fire_gathers(0, 0)
        fire_gathers(1, 1)

        def step(p, carry):
            a = 2 * p
            wait_gathers(0)
            fire_write(a, 0)
            wait_write(0)
            fire_gathers(a + 2, 0)
            wait_gathers(1)
            fire_write(a + 1, 1)
            wait_write(1)
            fire_gathers(a + 3, 1)
            return carry

        lax.fori_loop(0, groups // 2 - 1, step, 0)

        last = groups - 2
        wait_gathers(0)
        fire_write(last, 0)
        wait_gathers(1)
        fire_write(last + 1, 1)
        wait_write(0)
        wait_write(1)

    return body


def kernel(indices, table):
    tot = indices.shape[0] * indices.shape[1]
    rows_per_w = tot // (NW * CHUNK)
    idx = indices.reshape(NW, rows_per_w, CHUNK)
    out = _make_gather(rows_per_w, table.shape[0])(idx, table)
    return out.reshape(indices.shape[0], indices.shape[1], D)


# speedup vs baseline: 1.7991x; 1.6152x over previous
import functools

import jax
import jax.numpy as jnp
from jax import lax
from jax.experimental import pallas as pl
from jax.experimental.pallas import tpu as pltpu
from jax.experimental.pallas import tpu_sc as plsc

R = 8

_info = plsc.get_sparse_core_info()
NC, NS = _info.num_cores, _info.num_subcores
NW = NC * NS


@functools.lru_cache(maxsize=None)
def _make_gather(B: int, L: int, D: int, vocab: int):
    rows_w = B // NW
    groups = rows_w // R
    assert groups % 2 == 0 and groups >= 4
    mesh = plsc.VectorSubcoreMesh(core_axis_name="c", subcore_axis_name="s")

    @functools.partial(
        pl.kernel,
        mesh=mesh,
        out_type=jax.ShapeDtypeStruct((B, L, D), jnp.float32),
        scratch_types=[
            pltpu.VMEM((rows_w, L), jnp.int32),
            pltpu.VMEM((2, R, L, D), jnp.float32),
            pltpu.SemaphoreType.DMA,
            pltpu.SemaphoreType.DMA,
            pltpu.SemaphoreType.DMA,
            pltpu.SemaphoreType.DMA,
        ],
        compiler_params=pltpu.CompilerParams(use_tc_tiling_on_sc=False),
    )
    def body(idx_hbm, table_hbm, out_hbm, idx_v, rows_v, gs0, gs1, ws0, ws1):
        wid = lax.axis_index("s") * NC + lax.axis_index("c")
        base = wid * rows_w
        pltpu.sync_copy(idx_hbm.at[pl.ds(base, rows_w)], idx_v)

        gsem = (gs0, gs1)
        wsem = (ws0, ws1)

        def fire_gathers(g, b):
            for j in range(R):
                pltpu.async_copy(
                    table_hbm.at[idx_v.at[g * R + j]],
                    rows_v.at[b].at[j],
                    gsem[b])

        def wait_gathers(b):
            pltpu.make_async_copy(
                out_hbm.at[pl.ds(0, R)], rows_v.at[b], gsem[b]).wait()

        def fire_write(g, b):
            pltpu.async_copy(
                rows_v.at[b], out_hbm.at[pl.ds(base + g * R, R)], wsem[b])

        def wait_write(b):
            pltpu.make_async_copy(
                rows_v.at[b], out_hbm.at[pl.ds(0, R)], wsem[b]).wait()

        fire_gathers(0, 0)
        fire_gathers(1, 1)

        def step(p, carry):
            a = 2 * p
            wait_gathers(0)
            fire_write(a, 0)
            wait_write(0)
            fire_gathers(a + 2, 0)
            wait_gathers(1)
            fire_write(a + 1, 1)
            wait_write(1)
            fire_gathers(a + 3, 1)
            return carry

        lax.fori_loop(0, groups // 2 - 1, step, 0)

        last = groups - 2
        wait_gathers(0)
        fire_write(last, 0)
        wait_gathers(1)
        fire_write(last + 1, 1)
        wait_write(0)
        wait_write(1)

    return body


def kernel(indices, table):
    B, L = indices.shape
    return _make_gather(B, L, table.shape[1], table.shape[0])(indices, table)
